# Initial kernel scaffold; baseline (speedup 1.0000x reference)
import jax
import jax.numpy as jnp
from jax.experimental import pallas as pl


def kernel(x, adj, W1, b1, g1, beta1, W2, b2, g2, beta2, W3, b3, g3, beta3, fcW1, fcb1, fcW2, fcb2):
    raise NotImplementedError("write your pallas kernel here")


# speedup vs baseline: 1.0082x; 1.0082x over previous
import jax
import jax.numpy as jnp
from jax.experimental import pallas as pl
from jax.experimental.pallas import tpu as pltpu

F32 = jnp.float32
BF16 = jnp.bfloat16
_DIMS = (((1,), (0,)), ((), ()))


def _pick_bm(n):
    for bm in (400, 200, 100, 8):
        if n % bm == 0:
            return bm
    return n


def _support_body(x_ref, w_ref, o_ref):
    x16 = x_ref[...].astype(BF16)
    o_ref[...] = jax.lax.dot_general(
        x16, w_ref[...], _DIMS, preferred_element_type=F32).astype(BF16)


def _support(x, w16, bm):
    n, d = x.shape
    h = w16.shape[1]
    return pl.pallas_call(
        _support_body,
        grid=(n // bm,),
        in_specs=[
            pl.BlockSpec((bm, d), lambda m: (m, 0)),
            pl.BlockSpec((d, h), lambda m: (0, 0)),
        ],
        out_specs=pl.BlockSpec((bm, h), lambda m: (m, 0)),
        out_shape=jax.ShapeDtypeStruct((n, h), BF16),
        compiler_params=pltpu.CompilerParams(
            dimension_semantics=("parallel",)),
    )(x, w16)


def _ln_relu(h, g, beta):
    m = jnp.mean(h, axis=1, keepdims=True)
    c = h - m
    v = jnp.mean(c * c, axis=1, keepdims=True)
    y = c * jax.lax.rsqrt(v + 1e-5) * g + beta
    return jnp.maximum(y, 0.0)


def _layer_body(adj_ref, s_ref, b_ref, g_ref, beta_ref, wn_ref,
                x_ref, sn_ref):
    part = jax.lax.dot_general(
        adj_ref[...], s_ref[...], _DIMS, preferred_element_type=F32)
    xl = _ln_relu(part + b_ref[...], g_ref[...], beta_ref[...])
    x_ref[...] = xl
    sn_ref[...] = jax.lax.dot_general(
        xl.astype(BF16), wn_ref[...], _DIMS,
        preferred_element_type=F32).astype(BF16)


def _layer(adj16, s, b, g, beta, wn16, bm):
    n = adj16.shape[0]
    h = s.shape[1]
    small = pl.BlockSpec((1, h), lambda m: (0, 0))
    return pl.pallas_call(
        _layer_body,
        grid=(n // bm,),
        in_specs=[
            pl.BlockSpec((bm, n), lambda m: (m, 0)),
            pl.BlockSpec((n, h), lambda m: (0, 0)),
            small, small, small,
            pl.BlockSpec((h, h), lambda m: (0, 0)),
        ],
        out_specs=(
            pl.BlockSpec((bm, h), lambda m: (m, 0)),
            pl.BlockSpec((bm, h), lambda m: (m, 0)),
        ),
        out_shape=(
            jax.ShapeDtypeStruct((n, h), F32),
            jax.ShapeDtypeStruct((n, h), BF16),
        ),
        compiler_params=pltpu.CompilerParams(
            dimension_semantics=("parallel",)),
    )(adj16, s, b, g, beta, wn16)


def _final_body(adj_ref, s_ref, b_ref, g_ref, beta_ref,
                x1_ref, x2_ref, a_ref, bb_ref, c_ref, fcb1_ref,
                w2t_ref, fcb2_ref, o_ref):
    part = jax.lax.dot_general(
        adj_ref[...], s_ref[...], _DIMS, preferred_element_type=F32)
    x3 = _ln_relu(part + b_ref[...], g_ref[...], beta_ref[...])
    hf = jax.lax.dot_general(
        x1_ref[...].astype(BF16), a_ref[...], _DIMS,
        preferred_element_type=F32)
    hf += jax.lax.dot_general(
        x2_ref[...].astype(BF16), bb_ref[...], _DIMS,
        preferred_element_type=F32)
    hf += jax.lax.dot_general(
        x3.astype(BF16), c_ref[...], _DIMS, preferred_element_type=F32)
    hf = jnp.maximum(hf + fcb1_ref[...], 0.0)
    o = jnp.sum(hf * w2t_ref[...], axis=1, keepdims=True)
    o_ref[...] = o + fcb2_ref[...]


def _final(adj16, s, b, g, beta, x1, x2, a16, b16, c16, fcb1, w2t,
           fcb2, bm):
    n = adj16.shape[0]
    h = s.shape[1]
    small = pl.BlockSpec((1, h), lambda m: (0, 0))
    wspec = pl.BlockSpec((h, h), lambda m: (0, 0))
    xspec = pl.BlockSpec((bm, h), lambda m: (m, 0))
    return pl.pallas_call(
        _final_body,
        grid=(n // bm,),
        in_specs=[
            pl.BlockSpec((bm, n), lambda m: (m, 0)),
            pl.BlockSpec((n, h), lambda m: (0, 0)),
            small, small, small,
            xspec, xspec,
            wspec, wspec, wspec,
            small,
            small,
            pl.BlockSpec((1, 1), lambda m: (0, 0)),
        ],
        out_specs=pl.BlockSpec((bm, 1), lambda m: (m, 0)),
        out_shape=jax.ShapeDtypeStruct((n, 1), F32),
        compiler_params=pltpu.CompilerParams(
            dimension_semantics=("parallel",)),
    )(adj16, s, b, g, beta, x1, x2, a16, b16, c16, fcb1, w2t, fcb2)


def kernel(x, adj, W1, b1, g1, beta1, W2, b2, g2, beta2, W3, b3, g3,
           beta3, fcW1, fcb1, fcW2, fcb2):
    n, d = x.shape
    h = W1.shape[1]
    bm = _pick_bm(n)

    adj16 = adj.astype(BF16)
    row = lambda v: v.reshape(1, -1).astype(F32)
    s1 = _support(x, W1.astype(BF16), bm)
    x1, s2 = _layer(adj16, s1, row(b1), row(g1), row(beta1),
                    W2.astype(BF16), bm)
    x2, s3 = _layer(adj16, s2, row(b2), row(g2), row(beta2),
                    W3.astype(BF16), bm)
    a16 = fcW1[0:h].astype(BF16)
    b16 = fcW1[h:2 * h].astype(BF16)
    c16 = fcW1[2 * h:3 * h].astype(BF16)
    out = _final(adj16, s3, row(b3), row(g3), row(beta3), x1, x2,
                 a16, b16, c16, row(fcb1), fcW2.reshape(1, -1),
                 fcb2.reshape(1, 1), bm)
    return out.reshape(n)


# speedup vs baseline: 1.2072x; 1.1974x over previous
import jax
import jax.numpy as jnp
from jax.experimental import pallas as pl
from jax.experimental.pallas import tpu as pltpu

F32 = jnp.float32
BF16 = jnp.bfloat16
_DIMS = (((1,), (0,)), ((), ()))


def _pick_bm(n):
    for bm in (400, 200, 100, 8):
        if n % bm == 0:
            return bm
    return n


def _support_body(x_ref, w_ref, o_ref):
    x16 = x_ref[...].astype(BF16)
    o_ref[...] = jax.lax.dot_general(
        x16, w_ref[...], _DIMS, preferred_element_type=F32).astype(BF16)


def _support(x, w16, bm):
    n, d = x.shape
    h = w16.shape[1]
    return pl.pallas_call(
        _support_body,
        grid=(n // bm,),
        in_specs=[
            pl.BlockSpec((bm, d), lambda m: (m, 0)),
            pl.BlockSpec((d, h), lambda m: (0, 0)),
        ],
        out_specs=pl.BlockSpec((bm, h), lambda m: (m, 0)),
        out_shape=jax.ShapeDtypeStruct((n, h), BF16),
        compiler_params=pltpu.CompilerParams(
            dimension_semantics=("parallel",)),
    )(x, w16)


def _ln_relu(h, g, beta):
    m = jnp.mean(h, axis=1, keepdims=True)
    c = h - m
    v = jnp.mean(c * c, axis=1, keepdims=True)
    y = c * jax.lax.rsqrt(v + 1e-5) * g + beta
    return jnp.maximum(y, 0.0)


def _layer1_body(adj_ref, s_ref, b_ref, g_ref, beta_ref, wn_ref,
                 adj16_ref, x_ref, sn_ref):
    a16 = adj_ref[...].astype(BF16)
    adj16_ref[...] = a16
    part = jax.lax.dot_general(
        a16, s_ref[...], _DIMS, preferred_element_type=F32)
    xl = _ln_relu(part + b_ref[...], g_ref[...], beta_ref[...])
    x_ref[...] = xl.astype(BF16)
    sn_ref[...] = jax.lax.dot_general(
        xl.astype(BF16), wn_ref[...], _DIMS,
        preferred_element_type=F32).astype(BF16)


def _layer1(adj, s, b, g, beta, wn16, bm):
    n = adj.shape[0]
    h = s.shape[1]
    small = pl.BlockSpec((1, h), lambda m: (0, 0))
    return pl.pallas_call(
        _layer1_body,
        grid=(n // bm,),
        in_specs=[
            pl.BlockSpec((bm, n), lambda m: (m, 0)),
            pl.BlockSpec((n, h), lambda m: (0, 0)),
            small, small, small,
            pl.BlockSpec((h, h), lambda m: (0, 0)),
        ],
        out_specs=(
            pl.BlockSpec((bm, n), lambda m: (m, 0)),
            pl.BlockSpec((bm, h), lambda m: (m, 0)),
            pl.BlockSpec((bm, h), lambda m: (m, 0)),
        ),
        out_shape=(
            jax.ShapeDtypeStruct((n, n), BF16),
            jax.ShapeDtypeStruct((n, h), BF16),
            jax.ShapeDtypeStruct((n, h), BF16),
        ),
        compiler_params=pltpu.CompilerParams(
            dimension_semantics=("parallel",)),
    )(adj, s, b, g, beta, wn16)


def _layer_body(adj_ref, s_ref, b_ref, g_ref, beta_ref, wn_ref,
                x_ref, sn_ref):
    part = jax.lax.dot_general(
        adj_ref[...], s_ref[...], _DIMS, preferred_element_type=F32)
    xl = _ln_relu(part + b_ref[...], g_ref[...], beta_ref[...])
    x_ref[...] = xl.astype(BF16)
    sn_ref[...] = jax.lax.dot_general(
        xl.astype(BF16), wn_ref[...], _DIMS,
        preferred_element_type=F32).astype(BF16)


def _layer(adj16, s, b, g, beta, wn16, bm):
    n = adj16.shape[0]
    h = s.shape[1]
    small = pl.BlockSpec((1, h), lambda m: (0, 0))
    return pl.pallas_call(
        _layer_body,
        grid=(n // bm,),
        in_specs=[
            pl.BlockSpec((bm, n), lambda m: (m, 0)),
            pl.BlockSpec((n, h), lambda m: (0, 0)),
            small, small, small,
            pl.BlockSpec((h, h), lambda m: (0, 0)),
        ],
        out_specs=(
            pl.BlockSpec((bm, h), lambda m: (m, 0)),
            pl.BlockSpec((bm, h), lambda m: (m, 0)),
        ),
        out_shape=(
            jax.ShapeDtypeStruct((n, h), BF16),
            jax.ShapeDtypeStruct((n, h), BF16),
        ),
        compiler_params=pltpu.CompilerParams(
            dimension_semantics=("parallel",)),
    )(adj16, s, b, g, beta, wn16)


def _final_body(adj_ref, s_ref, b_ref, g_ref, beta_ref,
                x1_ref, x2_ref, a_ref, bb_ref, c_ref, fcb1_ref,
                w2t_ref, fcb2_ref, o_ref):
    part = jax.lax.dot_general(
        adj_ref[...], s_ref[...], _DIMS, preferred_element_type=F32)
    x3 = _ln_relu(part + b_ref[...], g_ref[...], beta_ref[...])
    hf = jax.lax.dot_general(
        x1_ref[...], a_ref[...], _DIMS, preferred_element_type=F32)
    hf += jax.lax.dot_general(
        x2_ref[...], bb_ref[...], _DIMS, preferred_element_type=F32)
    hf += jax.lax.dot_general(
        x3.astype(BF16), c_ref[...], _DIMS, preferred_element_type=F32)
    hf = jnp.maximum(hf + fcb1_ref[...], 0.0)
    o = jnp.sum(hf * w2t_ref[...], axis=1, keepdims=True)
    o_ref[...] = o + fcb2_ref[...]


def _final(adj16, s, b, g, beta, x1, x2, a16, b16, c16, fcb1, w2t,
           fcb2, bm):
    n = adj16.shape[0]
    h = s.shape[1]
    small = pl.BlockSpec((1, h), lambda m: (0, 0))
    wspec = pl.BlockSpec((h, h), lambda m: (0, 0))
    xspec = pl.BlockSpec((bm, h), lambda m: (m, 0))
    return pl.pallas_call(
        _final_body,
        grid=(n // bm,),
        in_specs=[
            pl.BlockSpec((bm, n), lambda m: (m, 0)),
            pl.BlockSpec((n, h), lambda m: (0, 0)),
            small, small, small,
            xspec, xspec,
            wspec, wspec, wspec,
            small,
            small,
            pl.BlockSpec((1, 1), lambda m: (0, 0)),
        ],
        out_specs=pl.BlockSpec((bm, 1), lambda m: (m, 0)),
        out_shape=jax.ShapeDtypeStruct((n, 1), F32),
        compiler_params=pltpu.CompilerParams(
            dimension_semantics=("parallel",)),
    )(adj16, s, b, g, beta, x1, x2, a16, b16, c16, fcb1, w2t, fcb2)


def kernel(x, adj, W1, b1, g1, beta1, W2, b2, g2, beta2, W3, b3, g3,
           beta3, fcW1, fcb1, fcW2, fcb2):
    n, d = x.shape
    h = W1.shape[1]
    bm = _pick_bm(n)

    row = lambda v: v.reshape(1, -1).astype(F32)
    s1 = _support(x, W1.astype(BF16), bm)
    adj16, x1, s2 = _layer1(adj, s1, row(b1), row(g1), row(beta1),
                            W2.astype(BF16), bm // 2)
    x2, s3 = _layer(adj16, s2, row(b2), row(g2), row(beta2),
                    W3.astype(BF16), bm)
    a16 = fcW1[0:h].astype(BF16)
    b16 = fcW1[h:2 * h].astype(BF16)
    c16 = fcW1[2 * h:3 * h].astype(BF16)
    out = _final(adj16, s3, row(b3), row(g3), row(beta3), x1, x2,
                 a16, b16, c16, row(fcb1), fcW2.reshape(1, -1),
                 fcb2.reshape(1, 1), bm)
    return out.reshape(n)


# speedup vs baseline: 1.2292x; 1.0183x over previous
import jax
import jax.numpy as jnp
from jax.experimental import pallas as pl
from jax.experimental.pallas import tpu as pltpu

F32 = jnp.float32
BF16 = jnp.bfloat16
_DIMS = (((1,), (0,)), ((), ()))


def _pick_bm(n):
    for bm in (400, 200, 100, 8):
        if n % bm == 0:
            return bm
    return n


def _support_body(x_ref, w_ref, o_ref):
    x16 = x_ref[...].astype(BF16)
    o_ref[...] = jax.lax.dot_general(
        x16, w_ref[...], _DIMS, preferred_element_type=F32).astype(BF16)


def _support(x, w16, bm):
    n, d = x.shape
    h = w16.shape[1]
    return pl.pallas_call(
        _support_body,
        grid=(n // bm,),
        in_specs=[
            pl.BlockSpec((bm, d), lambda m: (m, 0)),
            pl.BlockSpec((d, h), lambda m: (0, 0)),
        ],
        out_specs=pl.BlockSpec((bm, h), lambda m: (m, 0)),
        out_shape=jax.ShapeDtypeStruct((n, h), BF16),
        compiler_params=pltpu.CompilerParams(
            dimension_semantics=("parallel",)),
    )(x, w16)


def _ln_relu(h, g, beta):
    m = jnp.mean(h, axis=1, keepdims=True)
    c = h - m
    v = jnp.mean(c * c, axis=1, keepdims=True)
    y = c * jax.lax.rsqrt(v + 1e-5) * g + beta
    return jnp.maximum(y, 0.0)


def _layer1_body(adj_ref, s_ref, b_ref, g_ref, beta_ref, wn_ref,
                 adj16_ref, x_ref, sn_ref):
    a16 = adj_ref[...].astype(BF16)
    adj16_ref[...] = a16
    part = jax.lax.dot_general(
        a16, s_ref[...], _DIMS, preferred_element_type=F32)
    xl = _ln_relu(part + b_ref[...], g_ref[...], beta_ref[...])
    x_ref[...] = xl.astype(BF16)
    sn_ref[...] = jax.lax.dot_general(
        xl.astype(BF16), wn_ref[...], _DIMS,
        preferred_element_type=F32).astype(BF16)


def _layer1(adj, s, b, g, beta, wn16, bm):
    n = adj.shape[0]
    h = s.shape[1]
    small = pl.BlockSpec((1, h), lambda m: (0, 0))
    return pl.pallas_call(
        _layer1_body,
        grid=(n // bm,),
        in_specs=[
            pl.BlockSpec((bm, n), lambda m: (m, 0)),
            pl.BlockSpec((n, h), lambda m: (0, 0)),
            small, small, small,
            pl.BlockSpec((h, h), lambda m: (0, 0)),
        ],
        out_specs=(
            pl.BlockSpec((bm, n), lambda m: (m, 0)),
            pl.BlockSpec((bm, h), lambda m: (m, 0)),
            pl.BlockSpec((bm, h), lambda m: (m, 0)),
        ),
        out_shape=(
            jax.ShapeDtypeStruct((n, n), BF16),
            jax.ShapeDtypeStruct((n, h), BF16),
            jax.ShapeDtypeStruct((n, h), BF16),
        ),
        compiler_params=pltpu.CompilerParams(
            dimension_semantics=("parallel",)),
    )(adj, s, b, g, beta, wn16)


def _layer_body(adj_ref, s_ref, b_ref, g_ref, beta_ref, wn_ref,
                x_ref, sn_ref):
    part = jax.lax.dot_general(
        adj_ref[...], s_ref[...], _DIMS, preferred_element_type=F32)
    xl = _ln_relu(part + b_ref[...], g_ref[...], beta_ref[...])
    x_ref[...] = xl.astype(BF16)
    sn_ref[...] = jax.lax.dot_general(
        xl.astype(BF16), wn_ref[...], _DIMS,
        preferred_element_type=F32).astype(BF16)


def _layer(adj16, s, b, g, beta, wn16, bm):
    n = adj16.shape[0]
    h = s.shape[1]
    small = pl.BlockSpec((1, h), lambda m: (0, 0))
    return pl.pallas_call(
        _layer_body,
        grid=(n // bm,),
        in_specs=[
            pl.BlockSpec((bm, n), lambda m: (m, 0)),
            pl.BlockSpec((n, h), lambda m: (0, 0)),
            small, small, small,
            pl.BlockSpec((h, h), lambda m: (0, 0)),
        ],
        out_specs=(
            pl.BlockSpec((bm, h), lambda m: (m, 0)),
            pl.BlockSpec((bm, h), lambda m: (m, 0)),
        ),
        out_shape=(
            jax.ShapeDtypeStruct((n, h), BF16),
            jax.ShapeDtypeStruct((n, h), BF16),
        ),
        compiler_params=pltpu.CompilerParams(
            dimension_semantics=("parallel",),
            vmem_limit_bytes=100 * 1024 * 1024),
    )(adj16, s, b, g, beta, wn16)


def _final_body(adj_ref, s_ref, b_ref, g_ref, beta_ref,
                x1_ref, x2_ref, a_ref, bb_ref, c_ref, fcb1_ref,
                w2t_ref, fcb2_ref, o_ref):
    part = jax.lax.dot_general(
        adj_ref[...], s_ref[...], _DIMS, preferred_element_type=F32)
    x3 = _ln_relu(part + b_ref[...], g_ref[...], beta_ref[...])
    hf = jax.lax.dot_general(
        x1_ref[...], a_ref[...], _DIMS, preferred_element_type=F32)
    hf += jax.lax.dot_general(
        x2_ref[...], bb_ref[...], _DIMS, preferred_element_type=F32)
    hf += jax.lax.dot_general(
        x3.astype(BF16), c_ref[...], _DIMS, preferred_element_type=F32)
    hf = jnp.maximum(hf + fcb1_ref[...], 0.0)
    o = jnp.sum(hf * w2t_ref[...], axis=1, keepdims=True)
    o_ref[...] = o + fcb2_ref[...]


def _final(adj16, s, b, g, beta, x1, x2, a16, b16, c16, fcb1, w2t,
           fcb2, bm):
    n = adj16.shape[0]
    h = s.shape[1]
    small = pl.BlockSpec((1, h), lambda m: (0, 0))
    wspec = pl.BlockSpec((h, h), lambda m: (0, 0))
    xspec = pl.BlockSpec((bm, h), lambda m: (m, 0))
    return pl.pallas_call(
        _final_body,
        grid=(n // bm,),
        in_specs=[
            pl.BlockSpec((bm, n), lambda m: (m, 0)),
            pl.BlockSpec((n, h), lambda m: (0, 0)),
            small, small, small,
            xspec, xspec,
            wspec, wspec, wspec,
            small,
            small,
            pl.BlockSpec((1, 1), lambda m: (0, 0)),
        ],
        out_specs=pl.BlockSpec((bm, 1), lambda m: (m, 0)),
        out_shape=jax.ShapeDtypeStruct((n, 1), F32),
        compiler_params=pltpu.CompilerParams(
            dimension_semantics=("parallel",)),
    )(adj16, s, b, g, beta, x1, x2, a16, b16, c16, fcb1, w2t, fcb2)


def kernel(x, adj, W1, b1, g1, beta1, W2, b2, g2, beta2, W3, b3, g3,
           beta3, fcW1, fcb1, fcW2, fcb2):
    n, d = x.shape
    h = W1.shape[1]
    bm = _pick_bm(n)
    bml = 1000 if n % 1000 == 0 else bm

    row = lambda v: v.reshape(1, -1).astype(F32)
    s1 = _support(x, W1.astype(BF16), bm)
    adj16, x1, s2 = _layer1(adj, s1, row(b1), row(g1), row(beta1),
                            W2.astype(BF16), bm // 2)
    x2, s3 = _layer(adj16, s2, row(b2), row(g2), row(beta2),
                    W3.astype(BF16), bml)
    a16 = fcW1[0:h].astype(BF16)
    b16 = fcW1[h:2 * h].astype(BF16)
    c16 = fcW1[2 * h:3 * h].astype(BF16)
    out = _final(adj16, s3, row(b3), row(g3), row(beta3), x1, x2,
                 a16, b16, c16, row(fcb1), fcW2.reshape(1, -1),
                 fcb2.reshape(1, 1), bm)
    return out.reshape(n)


# speedup vs baseline: 1.2558x; 1.0216x over previous
import jax
import jax.numpy as jnp
from jax.experimental import pallas as pl
from jax.experimental.pallas import tpu as pltpu

F32 = jnp.float32
BF16 = jnp.bfloat16
_DIMS = (((1,), (0,)), ((), ()))


def _pick_bm(n):
    for bm in (400, 200, 100, 8):
        if n % bm == 0:
            return bm
    return n


def _support_body(x_ref, w_ref, o_ref):
    x16 = x_ref[...].astype(BF16)
    o_ref[...] = jax.lax.dot_general(
        x16, w_ref[...], _DIMS, preferred_element_type=F32).astype(BF16)


def _support(x, w16, bm):
    n, d = x.shape
    h = w16.shape[1]
    return pl.pallas_call(
        _support_body,
        grid=(n // bm,),
        in_specs=[
            pl.BlockSpec((bm, d), lambda m: (m, 0)),
            pl.BlockSpec((d, h), lambda m: (0, 0)),
        ],
        out_specs=pl.BlockSpec((bm, h), lambda m: (m, 0)),
        out_shape=jax.ShapeDtypeStruct((n, h), BF16),
        compiler_params=pltpu.CompilerParams(
            dimension_semantics=("parallel",)),
    )(x, w16)


def _ln_relu(h, g, beta):
    m = jnp.mean(h, axis=1, keepdims=True)
    c = h - m
    v = jnp.mean(c * c, axis=1, keepdims=True)
    y = c * jax.lax.rsqrt(v + 1e-5) * g + beta
    return jnp.maximum(y, 0.0)


def _layer1_body(adj_ref, s_ref, b_ref, g_ref, beta_ref, wn_ref,
                 adj16_ref, x_ref, sn_ref):
    a16 = adj_ref[...].astype(BF16)
    adj16_ref[...] = a16
    part = jax.lax.dot_general(
        a16, s_ref[...], _DIMS, preferred_element_type=F32)
    xl = _ln_relu(part + b_ref[...], g_ref[...], beta_ref[...])
    x_ref[...] = xl.astype(BF16)
    sn_ref[...] = jax.lax.dot_general(
        xl.astype(BF16), wn_ref[...], _DIMS,
        preferred_element_type=F32).astype(BF16)


def _layer1(adj, s, b, g, beta, wn16, bm):
    n = adj.shape[0]
    h = s.shape[1]
    small = pl.BlockSpec((1, h), lambda m: (0, 0))
    return pl.pallas_call(
        _layer1_body,
        grid=(n // bm,),
        in_specs=[
            pl.BlockSpec((bm, n), lambda m: (m, 0)),
            pl.BlockSpec((n, h), lambda m: (0, 0)),
            small, small, small,
            pl.BlockSpec((h, h), lambda m: (0, 0)),
        ],
        out_specs=(
            pl.BlockSpec((bm, n), lambda m: (m, 0)),
            pl.BlockSpec((bm, h), lambda m: (m, 0)),
            pl.BlockSpec((bm, h), lambda m: (m, 0)),
        ),
        out_shape=(
            jax.ShapeDtypeStruct((n, n), BF16),
            jax.ShapeDtypeStruct((n, h), BF16),
            jax.ShapeDtypeStruct((n, h), BF16),
        ),
        compiler_params=pltpu.CompilerParams(
            dimension_semantics=("parallel",),
            vmem_limit_bytes=100 * 1024 * 1024),
    )(adj, s, b, g, beta, wn16)


def _layer_body(adj_ref, s_ref, b_ref, g_ref, beta_ref, wn_ref,
                x_ref, sn_ref):
    part = jax.lax.dot_general(
        adj_ref[...], s_ref[...], _DIMS, preferred_element_type=F32)
    xl = _ln_relu(part + b_ref[...], g_ref[...], beta_ref[...])
    x_ref[...] = xl.astype(BF16)
    sn_ref[...] = jax.lax.dot_general(
        xl.astype(BF16), wn_ref[...], _DIMS,
        preferred_element_type=F32).astype(BF16)


def _layer(adj16, s, b, g, beta, wn16, bm):
    n = adj16.shape[0]
    h = s.shape[1]
    small = pl.BlockSpec((1, h), lambda m: (0, 0))
    return pl.pallas_call(
        _layer_body,
        grid=(n // bm,),
        in_specs=[
            pl.BlockSpec((bm, n), lambda m: (m, 0)),
            pl.BlockSpec((n, h), lambda m: (0, 0)),
            small, small, small,
            pl.BlockSpec((h, h), lambda m: (0, 0)),
        ],
        out_specs=(
            pl.BlockSpec((bm, h), lambda m: (m, 0)),
            pl.BlockSpec((bm, h), lambda m: (m, 0)),
        ),
        out_shape=(
            jax.ShapeDtypeStruct((n, h), BF16),
            jax.ShapeDtypeStruct((n, h), BF16),
        ),
        compiler_params=pltpu.CompilerParams(
            dimension_semantics=("parallel",),
            vmem_limit_bytes=100 * 1024 * 1024),
    )(adj16, s, b, g, beta, wn16)


def _final_body(adj_ref, s_ref, b_ref, g_ref, beta_ref,
                x1_ref, x2_ref, a_ref, bb_ref, c_ref, fcb1_ref,
                w2t_ref, fcb2_ref, o_ref):
    part = jax.lax.dot_general(
        adj_ref[...], s_ref[...], _DIMS, preferred_element_type=F32)
    x3 = _ln_relu(part + b_ref[...], g_ref[...], beta_ref[...])
    hf = jax.lax.dot_general(
        x1_ref[...], a_ref[...], _DIMS, preferred_element_type=F32)
    hf += jax.lax.dot_general(
        x2_ref[...], bb_ref[...], _DIMS, preferred_element_type=F32)
    hf += jax.lax.dot_general(
        x3.astype(BF16), c_ref[...], _DIMS, preferred_element_type=F32)
    hf = jnp.maximum(hf + fcb1_ref[...], 0.0)
    o = jnp.sum(hf * w2t_ref[...], axis=1, keepdims=True)
    o_ref[...] = o + fcb2_ref[...]


def _final(adj16, s, b, g, beta, x1, x2, a16, b16, c16, fcb1, w2t,
           fcb2, bm):
    n = adj16.shape[0]
    h = s.shape[1]
    small = pl.BlockSpec((1, h), lambda m: (0, 0))
    wspec = pl.BlockSpec((h, h), lambda m: (0, 0))
    xspec = pl.BlockSpec((bm, h), lambda m: (m, 0))
    return pl.pallas_call(
        _final_body,
        grid=(n // bm,),
        in_specs=[
            pl.BlockSpec((bm, n), lambda m: (m, 0)),
            pl.BlockSpec((n, h), lambda m: (0, 0)),
            small, small, small,
            xspec, xspec,
            wspec, wspec, wspec,
            small,
            small,
            pl.BlockSpec((1, 1), lambda m: (0, 0)),
        ],
        out_specs=pl.BlockSpec((bm, 1), lambda m: (m, 0)),
        out_shape=jax.ShapeDtypeStruct((n, 1), F32),
        compiler_params=pltpu.CompilerParams(
            dimension_semantics=("parallel",),
            vmem_limit_bytes=100 * 1024 * 1024),
    )(adj16, s, b, g, beta, x1, x2, a16, b16, c16, fcb1, w2t, fcb2)


def kernel(x, adj, W1, b1, g1, beta1, W2, b2, g2, beta2, W3, b3, g3,
           beta3, fcW1, fcb1, fcW2, fcb2):
    n, d = x.shape
    h = W1.shape[1]
    bm = _pick_bm(n)
    bml = 1000 if n % 1000 == 0 else bm

    row = lambda v: v.reshape(1, -1).astype(F32)
    s1 = _support(x, W1.astype(BF16), bm)
    adj16, x1, s2 = _layer1(adj, s1, row(b1), row(g1), row(beta1),
                            W2.astype(BF16), bm)
    x2, s3 = _layer(adj16, s2, row(b2), row(g2), row(beta2),
                    W3.astype(BF16), bml)
    a16 = fcW1[0:h].astype(BF16)
    b16 = fcW1[h:2 * h].astype(BF16)
    c16 = fcW1[2 * h:3 * h].astype(BF16)
    out = _final(adj16, s3, row(b3), row(g3), row(beta3), x1, x2,
                 a16, b16, c16, row(fcb1), fcW2.reshape(1, -1),
                 fcb2.reshape(1, 1), bml)
    return out.reshape(n)


# speedup vs baseline: 1.2954x; 1.0315x over previous
import jax
import jax.numpy as jnp
from jax.experimental import pallas as pl
from jax.experimental.pallas import tpu as pltpu

F32 = jnp.float32
BF16 = jnp.bfloat16
_DIMS = (((1,), (0,)), ((), ()))


def _pick_bm(n):
    for bm in (400, 200, 100, 8):
        if n % bm == 0:
            return bm
    return n


def _support_body(x_ref, w_ref, o_ref):
    x16 = x_ref[...].astype(BF16)
    o_ref[...] = jax.lax.dot_general(
        x16, w_ref[...], _DIMS, preferred_element_type=F32).astype(BF16)


def _support(x, w16, bm):
    n, d = x.shape
    h = w16.shape[1]
    return pl.pallas_call(
        _support_body,
        grid=(n // bm,),
        in_specs=[
            pl.BlockSpec((bm, d), lambda m: (m, 0)),
            pl.BlockSpec((d, h), lambda m: (0, 0)),
        ],
        out_specs=pl.BlockSpec((bm, h), lambda m: (m, 0)),
        out_shape=jax.ShapeDtypeStruct((n, h), BF16),
        compiler_params=pltpu.CompilerParams(
            dimension_semantics=("parallel",)),
    )(x, w16)


def _ln_relu(h, g, beta):
    m = jnp.mean(h, axis=1, keepdims=True)
    c = h - m
    v = jnp.mean(c * c, axis=1, keepdims=True)
    y = c * jax.lax.rsqrt(v + 1e-5) * g + beta
    return jnp.maximum(y, 0.0)


def _layer1_body(adj_ref, s_ref, b_ref, g_ref, beta_ref, wn_ref,
                 adj16_ref, x_ref, sn_ref):
    a16 = adj_ref[...].astype(BF16)
    adj16_ref[...] = a16
    part = jax.lax.dot_general(
        a16, s_ref[...], _DIMS, preferred_element_type=F32)
    xl = _ln_relu(part + b_ref[...], g_ref[...], beta_ref[...])
    x_ref[...] = xl.astype(BF16)
    sn_ref[...] = jax.lax.dot_general(
        xl.astype(BF16), wn_ref[...], _DIMS,
        preferred_element_type=F32).astype(BF16)


def _layer1(adj, s, b, g, beta, wn16, bm):
    n = adj.shape[0]
    h = s.shape[1]
    small = pl.BlockSpec((1, h), lambda m: (0, 0))
    return pl.pallas_call(
        _layer1_body,
        grid=(n // bm,),
        in_specs=[
            pl.BlockSpec((bm, n), lambda m: (m, 0)),
            pl.BlockSpec((n, h), lambda m: (0, 0)),
            small, small, small,
            pl.BlockSpec((h, h), lambda m: (0, 0)),
        ],
        out_specs=(
            pl.BlockSpec((bm, n), lambda m: (m, 0)),
            pl.BlockSpec((bm, h), lambda m: (m, 0)),
            pl.BlockSpec((bm, h), lambda m: (m, 0)),
        ),
        out_shape=(
            jax.ShapeDtypeStruct((n, n), BF16),
            jax.ShapeDtypeStruct((n, h), BF16),
            jax.ShapeDtypeStruct((n, h), BF16),
        ),
        compiler_params=pltpu.CompilerParams(
            dimension_semantics=("parallel",),
            vmem_limit_bytes=100 * 1024 * 1024),
    )(adj, s, b, g, beta, wn16)


def _layer_body(adj_ref, s_ref, b_ref, g_ref, beta_ref, wn_ref,
                x_ref, sn_ref):
    part = jax.lax.dot_general(
        adj_ref[...], s_ref[...], _DIMS, preferred_element_type=F32)
    xl = _ln_relu(part + b_ref[...], g_ref[...], beta_ref[...])
    x_ref[...] = xl.astype(BF16)
    sn_ref[...] = jax.lax.dot_general(
        xl.astype(BF16), wn_ref[...], _DIMS,
        preferred_element_type=F32).astype(BF16)


def _layer(adj16, s, b, g, beta, wn16, bm):
    n = adj16.shape[0]
    h = s.shape[1]
    small = pl.BlockSpec((1, h), lambda m: (0, 0))
    return pl.pallas_call(
        _layer_body,
        grid=(n // bm,),
        in_specs=[
            pl.BlockSpec((bm, n), lambda m: (m, 0)),
            pl.BlockSpec((n, h), lambda m: (0, 0)),
            small, small, small,
            pl.BlockSpec((h, h), lambda m: (0, 0)),
        ],
        out_specs=(
            pl.BlockSpec((bm, h), lambda m: (m, 0)),
            pl.BlockSpec((bm, h), lambda m: (m, 0)),
        ),
        out_shape=(
            jax.ShapeDtypeStruct((n, h), BF16),
            jax.ShapeDtypeStruct((n, h), BF16),
        ),
        compiler_params=pltpu.CompilerParams(
            dimension_semantics=("parallel",),
            vmem_limit_bytes=100 * 1024 * 1024),
    )(adj16, s, b, g, beta, wn16)


def _final_body(adj_ref, s_ref, b_ref, g_ref, beta_ref,
                x1_ref, x2_ref, a_ref, bb_ref, c_ref, fcb1_ref,
                w2t_ref, fcb2_ref, o_ref):
    part = jax.lax.dot_general(
        adj_ref[...], s_ref[...], _DIMS, preferred_element_type=F32)
    x3 = _ln_relu(part + b_ref[...], g_ref[...], beta_ref[...])
    hf = jax.lax.dot_general(
        x1_ref[...], a_ref[...], _DIMS, preferred_element_type=F32)
    hf += jax.lax.dot_general(
        x2_ref[...], bb_ref[...], _DIMS, preferred_element_type=F32)
    hf += jax.lax.dot_general(
        x3.astype(BF16), c_ref[...], _DIMS, preferred_element_type=F32)
    hf = jnp.maximum(hf + fcb1_ref[...], 0.0)
    o = jnp.sum(hf * w2t_ref[...], axis=1, keepdims=True)
    o_ref[...] = o + fcb2_ref[...]


def _final(adj16, s, b, g, beta, x1, x2, a16, b16, c16, fcb1, w2t,
           fcb2, bm):
    n = adj16.shape[0]
    h = s.shape[1]
    small = pl.BlockSpec((1, h), lambda m: (0, 0))
    wspec = pl.BlockSpec((h, h), lambda m: (0, 0))
    xspec = pl.BlockSpec((bm, h), lambda m: (m, 0))
    return pl.pallas_call(
        _final_body,
        grid=(n // bm,),
        in_specs=[
            pl.BlockSpec((bm, n), lambda m: (m, 0)),
            pl.BlockSpec((n, h), lambda m: (0, 0)),
            small, small, small,
            xspec, xspec,
            wspec, wspec, wspec,
            small,
            small,
            pl.BlockSpec((1, 1), lambda m: (0, 0)),
        ],
        out_specs=pl.BlockSpec((bm, 1), lambda m: (m, 0)),
        out_shape=jax.ShapeDtypeStruct((n, 1), F32),
        compiler_params=pltpu.CompilerParams(
            dimension_semantics=("parallel",),
            vmem_limit_bytes=100 * 1024 * 1024),
    )(adj16, s, b, g, beta, x1, x2, a16, b16, c16, fcb1, w2t, fcb2)


def _tail_body(adj_ref, s2_ref, b2_ref, g2_ref, bt2_ref, w3_ref,
               b3_ref, g3_ref, bt3_ref, x1_ref, a_ref, bb_ref, c_ref,
               fcb1_ref, w2t_ref, fcb2_ref, o_ref, x2_scr, s3_scr):
    l = pl.program_id(0)
    m = pl.program_id(1)

    @pl.when(l == 0)
    def _layer2():
        part = jax.lax.dot_general(
            adj_ref[...], s2_ref[...], _DIMS, preferred_element_type=F32)
        x2 = _ln_relu(part + b2_ref[...], g2_ref[...], bt2_ref[...])
        x2_scr[m] = x2.astype(BF16)
        s3_scr[m] = jax.lax.dot_general(
            x2.astype(BF16), w3_ref[...], _DIMS,
            preferred_element_type=F32).astype(BF16)

    @pl.when(l == 1)
    def _layer3_head():
        nb, bm, h = s3_scr.shape
        s3 = s3_scr[...].reshape(nb * bm, h)
        part = jax.lax.dot_general(
            adj_ref[...], s3, _DIMS, preferred_element_type=F32)
        x3 = _ln_relu(part + b3_ref[...], g3_ref[...], bt3_ref[...])
        hf = jax.lax.dot_general(
            x1_ref[...], a_ref[...], _DIMS, preferred_element_type=F32)
        hf += jax.lax.dot_general(
            x2_scr[m], bb_ref[...], _DIMS, preferred_element_type=F32)
        hf += jax.lax.dot_general(
            x3.astype(BF16), c_ref[...], _DIMS,
            preferred_element_type=F32)
        hf = jnp.maximum(hf + fcb1_ref[...], 0.0)
        o = jnp.sum(hf * w2t_ref[...], axis=1, keepdims=True)
        o_ref[...] = o + fcb2_ref[...]


def _tail(adj16, s2, b2, g2, bt2, w316, b3, g3, bt3, x1,
          a16, b16, c16, fcb1, w2t, fcb2, bm):
    n = adj16.shape[0]
    h = s2.shape[1]
    nb = n // bm
    small = pl.BlockSpec((1, h), lambda l, m: (0, 0))
    wspec = pl.BlockSpec((h, h), lambda l, m: (0, 0))
    return pl.pallas_call(
        _tail_body,
        grid=(2, nb),
        in_specs=[
            pl.BlockSpec((bm, n), lambda l, m: (m, 0)),
            pl.BlockSpec((n, h), lambda l, m: (0, 0)),
            small, small, small,
            wspec,
            small, small, small,
            pl.BlockSpec((bm, h),
                         lambda l, m: (jnp.where(l == 1, m, 0), 0)),
            wspec, wspec, wspec,
            small,
            small,
            pl.BlockSpec((1, 1), lambda l, m: (0, 0)),
        ],
        out_specs=pl.BlockSpec(
            (bm, 1), lambda l, m: (jnp.where(l == 1, m, 0), 0)),
        out_shape=jax.ShapeDtypeStruct((n, 1), F32),
        scratch_shapes=[
            pltpu.VMEM((nb, bm, h), BF16),
            pltpu.VMEM((nb, bm, h), BF16),
        ],
        compiler_params=pltpu.CompilerParams(
            dimension_semantics=("arbitrary", "arbitrary"),
            vmem_limit_bytes=100 * 1024 * 1024),
    )(adj16, s2, b2, g2, bt2, w316, b3, g3, bt3, x1,
      a16, b16, c16, fcb1, w2t, fcb2)


def kernel(x, adj, W1, b1, g1, beta1, W2, b2, g2, beta2, W3, b3, g3,
           beta3, fcW1, fcb1, fcW2, fcb2):
    n, d = x.shape
    h = W1.shape[1]
    bm = _pick_bm(n)
    bml = 1000 if n % 1000 == 0 else bm

    row = lambda v: v.reshape(1, -1).astype(F32)
    s1 = _support(x, W1.astype(BF16), bm)
    adj16, x1, s2 = _layer1(adj, s1, row(b1), row(g1), row(beta1),
                            W2.astype(BF16), bm)
    a16 = fcW1[0:h].astype(BF16)
    b16 = fcW1[h:2 * h].astype(BF16)
    c16 = fcW1[2 * h:3 * h].astype(BF16)
    out = _tail(adj16, s2, row(b2), row(g2), row(beta2),
                W3.astype(BF16), row(b3), row(g3), row(beta3), x1,
                a16, b16, c16, row(fcb1), fcW2.reshape(1, -1),
                fcb2.reshape(1, 1), bml)
    return out.reshape(n)
